# baseline (speedup 1.0000x reference)
import functools

import jax
import jax.numpy as jnp
from jax import lax
from jax.experimental import pallas as pl
from jax.experimental.pallas import tpu as pltpu
from jax.experimental.pallas import tpu_sc as plsc

D = 128
BATCH = 2
N = BATCH * D * D * D
NW = 32
PER_W = N // NW
C = 4096
NCH = PER_W // C
VPC = C // 16

OFFS = (0, 1, 128, 129, 16384, 16385, 16512, 16513)
SLAB = C + OFFS[-1] + 7
BSUB = 512
GSUB = 256


def _build_body(img_hbm, tab_hbm, slab_v, out8_v):
    cid = lax.axis_index("c")
    sid = lax.axis_index("s")
    wid = sid * 2 + cid
    lane = lax.broadcasted_iota(jnp.int32, (16,), 0)

    cols = [jnp.full((16,), k, jnp.int32) for k in range(8)]

    def chunk(ci, carry):
        s = wid * PER_W + ci * C
        s_eff = jnp.minimum(s, N - SLAB)
        delta = s - s_eff
        pltpu.sync_copy(img_hbm.at[pl.ds(s_eff, SLAB)], slab_v)

        for j in range(C // BSUB):
            @pl.when(delta == 0)
            def _fast():
                def loop(i, c1):
                    base = j * BSUB + i * 16
                    rows = i * 16 + lane
                    for k in range(8):
                        vals = slab_v[pl.ds(base + OFFS[k], 16)]
                        plsc.store_scatter(out8_v, [rows, cols[k]], vals)
                    return c1

                lax.fori_loop(0, BSUB // 16, loop, 0)

            @pl.when(delta != 0)
            def _clamped():
                def loop(i, c1):
                    base = delta + j * BSUB + i * 16
                    rows = i * 16 + lane
                    for k in range(8):
                        iv = jnp.minimum(base + OFFS[k] + lane, SLAB - 1)
                        vals = plsc.load_gather(slab_v, [iv])
                        plsc.store_scatter(out8_v, [rows, cols[k]], vals)
                    return c1

                lax.fori_loop(0, BSUB // 16, loop, 0)

            pltpu.sync_copy(out8_v, tab_hbm.at[pl.ds(s + j * BSUB, BSUB), :])
        return carry

    lax.fori_loop(0, NCH, chunk, 0)


_build = functools.partial(
    pl.kernel,
    out_type=jax.ShapeDtypeStruct((N, 8), jnp.float32),
    mesh=plsc.VectorSubcoreMesh(core_axis_name="c", subcore_axis_name="s"),
    scratch_types=[
        pltpu.VMEM((SLAB,), jnp.float32),
        pltpu.VMEM((BSUB, 8), jnp.float32),
    ],
    compiler_params=pltpu.CompilerParams(needs_layout_passes=False),
)(_build_body)


def _warp_body(ddf0_hbm, ddf1_hbm, ddf2_hbm, tab_hbm, out_hbm,
               d0_v, d1_v, d2_v, g_v, wb_v, vals_v, out_v, gsem):
    cid = lax.axis_index("c")
    sid = lax.axis_index("s")
    wid = sid * 2 + cid
    lane = lax.broadcasted_iota(jnp.int32, (16,), 0)

    def chunk(ci, carry):
        vb = wid * PER_W + ci * C

        pltpu.sync_copy(ddf0_hbm.at[pl.ds(vb, C)], d0_v)
        pltpu.sync_copy(ddf1_hbm.at[pl.ds(vb, C)], d1_v)
        pltpu.sync_copy(ddf2_hbm.at[pl.ds(vb, C)], d2_v)

        def pass1(i, carry1):
            off = i * 16
            v = vb + off + lane
            b = lax.shift_right_logical(v, 21)
            n = jnp.bitwise_and(v, (1 << 21) - 1)
            x = lax.shift_right_logical(n, 14)
            y = jnp.bitwise_and(lax.shift_right_logical(n, 7), 127)
            z = jnp.bitwise_and(n, 127)
            cx = x.astype(jnp.float32) + d0_v[pl.ds(off, 16)]
            cy = y.astype(jnp.float32) + d1_v[pl.ds(off, 16)]
            cz = z.astype(jnp.float32) + d2_v[pl.ds(off, 16)]
            tx = cx.astype(jnp.int32)
            ty = cy.astype(jnp.int32)
            tz = cz.astype(jnp.int32)
            fx = jnp.where(tx.astype(jnp.float32) > cx, tx - 1, tx)
            fy = jnp.where(ty.astype(jnp.float32) > cy, ty - 1, ty)
            fz = jnp.where(tz.astype(jnp.float32) > cz, tz - 1, tz)
            bx = jnp.logical_or(cx >= jnp.float32(D - 1), cx < jnp.float32(0))
            by = jnp.logical_or(cy >= jnp.float32(D - 1), cy < jnp.float32(0))
            bz = jnp.logical_or(cz >= jnp.float32(D - 1), cz < jnp.float32(0))
            zero = jnp.zeros((16,), jnp.float32)
            wb_v[0, pl.ds(off, 16)] = jnp.where(bx, zero, cx - fx.astype(jnp.float32))
            wb_v[1, pl.ds(off, 16)] = jnp.where(by, zero, cy - fy.astype(jnp.float32))
            wb_v[2, pl.ds(off, 16)] = jnp.where(bz, zero, cz - fz.astype(jnp.float32))
            x0 = jnp.clip(fx, 0, D - 1)
            y0 = jnp.clip(fy, 0, D - 1)
            z0 = jnp.clip(fz, 0, D - 1)
            g_v[pl.ds(off, 16)] = (
                lax.shift_left(b, 21)
                + lax.shift_left(x0, 14)
                + lax.shift_left(y0, 7)
                + z0
            )
            return carry1

        lax.fori_loop(0, VPC, pass1, 0)

        cols = [jnp.full((16,), k, jnp.int32) for k in range(8)]
        for j in range(C // GSUB):
            pltpu.async_copy(
                tab_hbm.at[g_v.at[pl.ds(j * GSUB, GSUB)]], vals_v, gsem
            ).wait()

            def pass2(i, carry2):
                off = j * GSUB + i * 16
                vid = i * 16 + lane
                wxb = wb_v[0, pl.ds(off, 16)]
                wyb = wb_v[1, pl.ds(off, 16)]
                wzb = wb_v[2, pl.ds(off, 16)]
                wxa = 1.0 - wxb
                wya = 1.0 - wyb
                wza = 1.0 - wzb
                w00 = wxa * wya
                w01 = wxa * wyb
                w10 = wxb * wya
                w11 = wxb * wyb
                acc = plsc.load_gather(vals_v, [vid, cols[0]]) * (w00 * wza)
                acc = acc + plsc.load_gather(vals_v, [vid, cols[1]]) * (w00 * wzb)
                acc = acc + plsc.load_gather(vals_v, [vid, cols[2]]) * (w01 * wza)
                acc = acc + plsc.load_gather(vals_v, [vid, cols[3]]) * (w01 * wzb)
                acc = acc + plsc.load_gather(vals_v, [vid, cols[4]]) * (w10 * wza)
                acc = acc + plsc.load_gather(vals_v, [vid, cols[5]]) * (w10 * wzb)
                acc = acc + plsc.load_gather(vals_v, [vid, cols[6]]) * (w11 * wza)
                acc = acc + plsc.load_gather(vals_v, [vid, cols[7]]) * (w11 * wzb)
                out_v[pl.ds(off, 16)] = acc
                return carry2

            lax.fori_loop(0, GSUB // 16, pass2, 0)

        pltpu.sync_copy(out_v, out_hbm.at[pl.ds(vb, C)])
        return carry

    lax.fori_loop(0, NCH, chunk, 0)


_warp = functools.partial(
    pl.kernel,
    out_type=jax.ShapeDtypeStruct((N,), jnp.float32),
    mesh=plsc.VectorSubcoreMesh(core_axis_name="c", subcore_axis_name="s"),
    scratch_types=[
        pltpu.VMEM((C,), jnp.float32),
        pltpu.VMEM((C,), jnp.float32),
        pltpu.VMEM((C,), jnp.float32),
        pltpu.VMEM((C,), jnp.int32),
        pltpu.VMEM((3, C), jnp.float32),
        pltpu.VMEM((GSUB, 8), jnp.float32),
        pltpu.VMEM((C,), jnp.float32),
        pltpu.SemaphoreType.DMA,
    ],
    compiler_params=pltpu.CompilerParams(
        needs_layout_passes=False, use_tc_tiling_on_sc=False),
)(_warp_body)


def kernel(ddf, image):
    d0 = ddf[..., 0].reshape(-1)
    d1 = ddf[..., 1].reshape(-1)
    d2 = ddf[..., 2].reshape(-1)
    tab = _build(image.reshape(-1))
    out_flat = _warp(d0, d1, d2, tab)
    return out_flat.reshape(BATCH, D, D, D)


# speedup vs baseline: 2.9717x; 2.9717x over previous
import functools

import jax
import jax.numpy as jnp
from jax import lax
from jax.experimental import pallas as pl
from jax.experimental.pallas import tpu as pltpu
from jax.experimental.pallas import tpu_sc as plsc

D = 128
BATCH = 2
N = BATCH * D * D * D
NW = 32
PER_W = N // NW
C = 4096
NCH = PER_W // C
VPC = C // 16

OFFS = (0, 1, 128, 129, 16384, 16385, 16512, 16513)
SLAB = C + OFFS[-1] + 7
BSUB = 512
GSUB = 256


def _build_body(img_hbm, tab_hbm, slab_v, out8_v):
    cid = lax.axis_index("c")
    sid = lax.axis_index("s")
    wid = sid * 2 + cid
    lane = lax.broadcasted_iota(jnp.int32, (16,), 0)

    def chunk(ci, carry):
        s = wid * PER_W + ci * C
        s_eff = jnp.minimum(s, N - SLAB)
        delta = s - s_eff
        pltpu.sync_copy(img_hbm.at[pl.ds(s_eff, SLAB)], slab_v)

        for j in range(C // BSUB):
            @pl.when(delta == 0)
            def _fast():
                def loop(i, c1):
                    base = j * BSUB + i * 16
                    flat = (i * 16 + lane) * 8
                    for k in range(8):
                        vals = slab_v[pl.ds(base + OFFS[k], 16)]
                        plsc.store_scatter(out8_v, [flat + k], vals)
                    return c1

                lax.fori_loop(0, BSUB // 16, loop, 0)

            @pl.when(delta != 0)
            def _clamped():
                def loop(i, c1):
                    base = delta + j * BSUB + i * 16
                    flat = (i * 16 + lane) * 8
                    for k in range(8):
                        iv = jnp.minimum(base + OFFS[k] + lane, SLAB - 1)
                        vals = plsc.load_gather(slab_v, [iv])
                        plsc.store_scatter(out8_v, [flat + k], vals)
                    return c1

                lax.fori_loop(0, BSUB // 16, loop, 0)

            pltpu.sync_copy(out8_v, tab_hbm.at[pl.ds((s + j * BSUB) * 8, BSUB * 8)])
        return carry

    lax.fori_loop(0, NCH, chunk, 0)


_build = functools.partial(
    pl.kernel,
    out_type=jax.ShapeDtypeStruct((N * 8,), jnp.float32),
    mesh=plsc.VectorSubcoreMesh(core_axis_name="c", subcore_axis_name="s"),
    scratch_types=[
        pltpu.VMEM((SLAB,), jnp.float32),
        pltpu.VMEM((BSUB * 8,), jnp.float32),
    ],
    compiler_params=pltpu.CompilerParams(needs_layout_passes=False),
)(_build_body)


def _warp_body(ddf0_hbm, ddf1_hbm, ddf2_hbm, tab_hbm, out_hbm,
               d0_v, d1_v, d2_v, g_v, wb_v, vals_v, out_v, gsem):
    cid = lax.axis_index("c")
    sid = lax.axis_index("s")
    wid = sid * 2 + cid
    lane = lax.broadcasted_iota(jnp.int32, (16,), 0)

    def chunk(ci, carry):
        vb = wid * PER_W + ci * C

        pltpu.sync_copy(ddf0_hbm.at[pl.ds(vb, C)], d0_v)
        pltpu.sync_copy(ddf1_hbm.at[pl.ds(vb, C)], d1_v)
        pltpu.sync_copy(ddf2_hbm.at[pl.ds(vb, C)], d2_v)

        def pass1(i, carry1):
            off = i * 16
            v = vb + off + lane
            b = lax.shift_right_logical(v, 21)
            n = jnp.bitwise_and(v, (1 << 21) - 1)
            x = lax.shift_right_logical(n, 14)
            y = jnp.bitwise_and(lax.shift_right_logical(n, 7), 127)
            z = jnp.bitwise_and(n, 127)
            cx = x.astype(jnp.float32) + d0_v[pl.ds(off, 16)]
            cy = y.astype(jnp.float32) + d1_v[pl.ds(off, 16)]
            cz = z.astype(jnp.float32) + d2_v[pl.ds(off, 16)]
            tx = cx.astype(jnp.int32)
            ty = cy.astype(jnp.int32)
            tz = cz.astype(jnp.int32)
            fx = jnp.where(tx.astype(jnp.float32) > cx, tx - 1, tx)
            fy = jnp.where(ty.astype(jnp.float32) > cy, ty - 1, ty)
            fz = jnp.where(tz.astype(jnp.float32) > cz, tz - 1, tz)
            bx = jnp.logical_or(cx >= jnp.float32(D - 1), cx < jnp.float32(0))
            by = jnp.logical_or(cy >= jnp.float32(D - 1), cy < jnp.float32(0))
            bz = jnp.logical_or(cz >= jnp.float32(D - 1), cz < jnp.float32(0))
            zero = jnp.zeros((16,), jnp.float32)
            wb_v[0, pl.ds(off, 16)] = jnp.where(bx, zero, cx - fx.astype(jnp.float32))
            wb_v[1, pl.ds(off, 16)] = jnp.where(by, zero, cy - fy.astype(jnp.float32))
            wb_v[2, pl.ds(off, 16)] = jnp.where(bz, zero, cz - fz.astype(jnp.float32))
            x0 = jnp.clip(fx, 0, D - 1)
            y0 = jnp.clip(fy, 0, D - 1)
            z0 = jnp.clip(fz, 0, D - 1)
            g_v[pl.ds(off, 16)] = (
                lax.shift_left(b, 21)
                + lax.shift_left(x0, 14)
                + lax.shift_left(y0, 7)
                + z0
            )
            return carry1

        lax.fori_loop(0, VPC, pass1, 0)

        cols = [jnp.full((16,), k, jnp.int32) for k in range(8)]
        for j in range(C // GSUB):
            pltpu.async_copy(
                tab_hbm.at[g_v.at[pl.ds(j * GSUB, GSUB)]], vals_v, gsem
            ).wait()

            def pass2(i, carry2):
                off = j * GSUB + i * 16
                vid = i * 16 + lane
                wxb = wb_v[0, pl.ds(off, 16)]
                wyb = wb_v[1, pl.ds(off, 16)]
                wzb = wb_v[2, pl.ds(off, 16)]
                wxa = 1.0 - wxb
                wya = 1.0 - wyb
                wza = 1.0 - wzb
                w00 = wxa * wya
                w01 = wxa * wyb
                w10 = wxb * wya
                w11 = wxb * wyb
                acc = plsc.load_gather(vals_v, [vid, cols[0]]) * (w00 * wza)
                acc = acc + plsc.load_gather(vals_v, [vid, cols[1]]) * (w00 * wzb)
                acc = acc + plsc.load_gather(vals_v, [vid, cols[2]]) * (w01 * wza)
                acc = acc + plsc.load_gather(vals_v, [vid, cols[3]]) * (w01 * wzb)
                acc = acc + plsc.load_gather(vals_v, [vid, cols[4]]) * (w10 * wza)
                acc = acc + plsc.load_gather(vals_v, [vid, cols[5]]) * (w10 * wzb)
                acc = acc + plsc.load_gather(vals_v, [vid, cols[6]]) * (w11 * wza)
                acc = acc + plsc.load_gather(vals_v, [vid, cols[7]]) * (w11 * wzb)
                out_v[pl.ds(off, 16)] = acc
                return carry2

            lax.fori_loop(0, GSUB // 16, pass2, 0)

        pltpu.sync_copy(out_v, out_hbm.at[pl.ds(vb, C)])
        return carry

    lax.fori_loop(0, NCH, chunk, 0)


_warp = functools.partial(
    pl.kernel,
    out_type=jax.ShapeDtypeStruct((N,), jnp.float32),
    mesh=plsc.VectorSubcoreMesh(core_axis_name="c", subcore_axis_name="s"),
    scratch_types=[
        pltpu.VMEM((C,), jnp.float32),
        pltpu.VMEM((C,), jnp.float32),
        pltpu.VMEM((C,), jnp.float32),
        pltpu.VMEM((C,), jnp.int32),
        pltpu.VMEM((3, C), jnp.float32),
        pltpu.VMEM((GSUB, 8), jnp.float32),
        pltpu.VMEM((C,), jnp.float32),
        pltpu.SemaphoreType.DMA,
    ],
    compiler_params=pltpu.CompilerParams(
        needs_layout_passes=False, use_tc_tiling_on_sc=False),
)(_warp_body)


def kernel(ddf, image):
    d0 = ddf[..., 0].reshape(-1)
    d1 = ddf[..., 1].reshape(-1)
    d2 = ddf[..., 2].reshape(-1)
    tab = _build(image.reshape(-1))
    out_flat = _warp(d0, d1, d2, tab.reshape(N, 8))
    return out_flat.reshape(BATCH, D, D, D)


# speedup vs baseline: 3.7301x; 1.2552x over previous
import functools

import jax
import jax.numpy as jnp
from jax import lax
from jax.experimental import pallas as pl
from jax.experimental.pallas import tpu as pltpu
from jax.experimental.pallas import tpu_sc as plsc

D = 128
BATCH = 2
N = BATCH * D * D * D
NW = 32
PER_W = N // NW
C = 4096
NCH = PER_W // C
VPC = C // 16

OFFS = (0, 1, 128, 129, 16384, 16385, 16512, 16513)
SLAB = C + OFFS[-1] + 7
BSUB = 512
GSUB = 256


def _build_body(img_hbm, tab_hbm, slab_v, out8_v):
    cid = lax.axis_index("c")
    sid = lax.axis_index("s")
    wid = sid * 2 + cid
    lane = lax.broadcasted_iota(jnp.int32, (16,), 0)

    def chunk(ci, carry):
        s = wid * PER_W + ci * C
        s_eff = jnp.minimum(s, N - SLAB)
        delta = s - s_eff
        pltpu.sync_copy(img_hbm.at[pl.ds(s_eff, SLAB)], slab_v)

        for j in range(C // BSUB):
            @pl.when(delta == 0)
            def _fast():
                def loop(i, c1):
                    base = j * BSUB + i * 16
                    flat = (i * 16 + lane) * 8
                    for k in range(8):
                        vals = slab_v[pl.ds(base + OFFS[k], 16)]
                        plsc.store_scatter(out8_v, [flat + k], vals)
                    return c1

                lax.fori_loop(0, BSUB // 16, loop, 0)

            @pl.when(delta != 0)
            def _clamped():
                def loop(i, c1):
                    base = delta + j * BSUB + i * 16
                    flat = (i * 16 + lane) * 8
                    for k in range(8):
                        iv = jnp.minimum(base + OFFS[k] + lane, SLAB - 1)
                        vals = plsc.load_gather(slab_v, [iv])
                        plsc.store_scatter(out8_v, [flat + k], vals)
                    return c1

                lax.fori_loop(0, BSUB // 16, loop, 0)

            pltpu.sync_copy(out8_v, tab_hbm.at[pl.ds((s + j * BSUB) * 8, BSUB * 8)])
        return carry

    lax.fori_loop(0, NCH, chunk, 0)


_build = functools.partial(
    pl.kernel,
    out_type=jax.ShapeDtypeStruct((N * 8,), jnp.float32),
    mesh=plsc.VectorSubcoreMesh(core_axis_name="c", subcore_axis_name="s"),
    scratch_types=[
        pltpu.VMEM((SLAB,), jnp.float32),
        pltpu.VMEM((BSUB * 8,), jnp.float32),
    ],
    compiler_params=pltpu.CompilerParams(needs_layout_passes=False),
)(_build_body)


def _warp_body(ddf0_hbm, ddf1_hbm, ddf2_hbm, tab_hbm, out_hbm,
               d0_v, d1_v, d2_v, g_v, wb_v, vals_a, vals_b, out_v, gsem, osem):
    cid = lax.axis_index("c")
    sid = lax.axis_index("s")
    wid = sid * 2 + cid
    lane = lax.broadcasted_iota(jnp.int32, (16,), 0)

    def chunk(ci, carry):
        vb = wid * PER_W + ci * C

        pltpu.sync_copy(ddf0_hbm.at[pl.ds(vb, C)], d0_v)
        pltpu.sync_copy(ddf1_hbm.at[pl.ds(vb, C)], d1_v)
        pltpu.sync_copy(ddf2_hbm.at[pl.ds(vb, C)], d2_v)

        def pass1(i, carry1):
            off = i * 16
            v = vb + off + lane
            b = lax.shift_right_logical(v, 21)
            n = jnp.bitwise_and(v, (1 << 21) - 1)
            x = lax.shift_right_logical(n, 14)
            y = jnp.bitwise_and(lax.shift_right_logical(n, 7), 127)
            z = jnp.bitwise_and(n, 127)
            cx = x.astype(jnp.float32) + d0_v[pl.ds(off, 16)]
            cy = y.astype(jnp.float32) + d1_v[pl.ds(off, 16)]
            cz = z.astype(jnp.float32) + d2_v[pl.ds(off, 16)]
            tx = cx.astype(jnp.int32)
            ty = cy.astype(jnp.int32)
            tz = cz.astype(jnp.int32)
            fx = jnp.where(tx.astype(jnp.float32) > cx, tx - 1, tx)
            fy = jnp.where(ty.astype(jnp.float32) > cy, ty - 1, ty)
            fz = jnp.where(tz.astype(jnp.float32) > cz, tz - 1, tz)
            bx = jnp.logical_or(cx >= jnp.float32(D - 1), cx < jnp.float32(0))
            by = jnp.logical_or(cy >= jnp.float32(D - 1), cy < jnp.float32(0))
            bz = jnp.logical_or(cz >= jnp.float32(D - 1), cz < jnp.float32(0))
            zero = jnp.zeros((16,), jnp.float32)
            wb_v[0, pl.ds(off, 16)] = jnp.where(bx, zero, cx - fx.astype(jnp.float32))
            wb_v[1, pl.ds(off, 16)] = jnp.where(by, zero, cy - fy.astype(jnp.float32))
            wb_v[2, pl.ds(off, 16)] = jnp.where(bz, zero, cz - fz.astype(jnp.float32))
            x0 = jnp.clip(fx, 0, D - 1)
            y0 = jnp.clip(fy, 0, D - 1)
            z0 = jnp.clip(fz, 0, D - 1)
            g_v[pl.ds(off, 16)] = (
                lax.shift_left(b, 21)
                + lax.shift_left(x0, 14)
                + lax.shift_left(y0, 7)
                + z0
            )
            return carry1

        lax.fori_loop(0, VPC, pass1, 0)

        @pl.when(ci > 0)
        def _drain_out():
            pltpu.make_async_copy(out_v, out_hbm.at[pl.ds(vb, C)], osem).wait()

        cols = [jnp.full((16,), k, jnp.int32) for k in range(8)]

        def pass2_sub(j, buf):
            def pass2(i, carry2):
                off = j * GSUB + i * 16
                vid = i * 16 + lane
                wxb = wb_v[0, pl.ds(off, 16)]
                wyb = wb_v[1, pl.ds(off, 16)]
                wzb = wb_v[2, pl.ds(off, 16)]
                wxa = 1.0 - wxb
                wya = 1.0 - wyb
                wza = 1.0 - wzb
                w00 = wxa * wya
                w01 = wxa * wyb
                w10 = wxb * wya
                w11 = wxb * wyb
                acc = plsc.load_gather(buf, [vid, cols[0]]) * (w00 * wza)
                acc = acc + plsc.load_gather(buf, [vid, cols[1]]) * (w00 * wzb)
                acc = acc + plsc.load_gather(buf, [vid, cols[2]]) * (w01 * wza)
                acc = acc + plsc.load_gather(buf, [vid, cols[3]]) * (w01 * wzb)
                acc = acc + plsc.load_gather(buf, [vid, cols[4]]) * (w10 * wza)
                acc = acc + plsc.load_gather(buf, [vid, cols[5]]) * (w10 * wzb)
                acc = acc + plsc.load_gather(buf, [vid, cols[6]]) * (w11 * wza)
                acc = acc + plsc.load_gather(buf, [vid, cols[7]]) * (w11 * wzb)
                out_v[pl.ds(off, 16)] = acc
                return carry2

            lax.fori_loop(0, GSUB // 16, pass2, 0)

        def fire(j, buf):
            return pltpu.async_copy(
                tab_hbm.at[g_v.at[pl.ds(j * GSUB, GSUB)]], buf, gsem
            )

        bufs = (vals_a, vals_b)
        subs = C // GSUB
        cp_prev = fire(0, bufs[0])
        for j in range(1, subs):
            cp = fire(j, bufs[j % 2])
            cp_prev.wait()
            pass2_sub(j - 1, bufs[(j - 1) % 2])
            cp_prev = cp
        cp_prev.wait()
        pass2_sub(subs - 1, bufs[(subs - 1) % 2])

        pltpu.async_copy(out_v, out_hbm.at[pl.ds(vb, C)], osem)
        return carry

    lax.fori_loop(0, NCH, chunk, 0)

    last = wid * PER_W + (NCH - 1) * C
    pltpu.make_async_copy(out_v, out_hbm.at[pl.ds(last, C)], osem).wait()


_warp = functools.partial(
    pl.kernel,
    out_type=jax.ShapeDtypeStruct((N,), jnp.float32),
    mesh=plsc.VectorSubcoreMesh(core_axis_name="c", subcore_axis_name="s"),
    scratch_types=[
        pltpu.VMEM((C,), jnp.float32),
        pltpu.VMEM((C,), jnp.float32),
        pltpu.VMEM((C,), jnp.float32),
        pltpu.VMEM((C,), jnp.int32),
        pltpu.VMEM((3, C), jnp.float32),
        pltpu.VMEM((GSUB, 8), jnp.float32),
        pltpu.VMEM((GSUB, 8), jnp.float32),
        pltpu.VMEM((C,), jnp.float32),
        pltpu.SemaphoreType.DMA,
        pltpu.SemaphoreType.DMA,
    ],
    compiler_params=pltpu.CompilerParams(
        needs_layout_passes=False, use_tc_tiling_on_sc=False),
)(_warp_body)


def kernel(ddf, image):
    d0 = ddf[..., 0].reshape(-1)
    d1 = ddf[..., 1].reshape(-1)
    d2 = ddf[..., 2].reshape(-1)
    tab = _build(image.reshape(-1))
    out_flat = _warp(d0, d1, d2, tab.reshape(N, 8))
    return out_flat.reshape(BATCH, D, D, D)


# speedup vs baseline: 4.0002x; 1.0724x over previous
import functools

import jax
import jax.numpy as jnp
from jax import lax
from jax.experimental import pallas as pl
from jax.experimental.pallas import tpu as pltpu
from jax.experimental.pallas import tpu_sc as plsc

D = 128
BATCH = 2
N = BATCH * D * D * D
NW = 32
PER_W = N // NW
C = 4096
NCH = PER_W // C
VPC = C // 16

OFFS = (0, 1, 128, 129, 16384, 16385, 16512, 16513)
SLAB = C + OFFS[-1] + 7
BSUB = 512
GSUB = 256


def _build_body(img_hbm, tab_hbm, slab_v, out8_v):
    cid = lax.axis_index("c")
    sid = lax.axis_index("s")
    wid = sid * 2 + cid
    lane = lax.broadcasted_iota(jnp.int32, (16,), 0)

    def chunk(ci, carry):
        s = wid * PER_W + ci * C
        s_eff = jnp.minimum(s, N - SLAB)
        delta = s - s_eff
        pltpu.sync_copy(img_hbm.at[pl.ds(s_eff, SLAB)], slab_v)

        for j in range(C // BSUB):
            @pl.when(delta == 0)
            def _fast():
                def loop(i, c1):
                    base = j * BSUB + i * 16
                    flat = (i * 16 + lane) * 8
                    for k in range(8):
                        vals = slab_v[pl.ds(base + OFFS[k], 16)]
                        plsc.store_scatter(out8_v, [flat + k], vals)
                    return c1

                lax.fori_loop(0, BSUB // 16, loop, 0)

            @pl.when(delta != 0)
            def _clamped():
                def loop(i, c1):
                    base = delta + j * BSUB + i * 16
                    flat = (i * 16 + lane) * 8
                    for k in range(8):
                        iv = jnp.minimum(base + OFFS[k] + lane, SLAB - 1)
                        vals = plsc.load_gather(slab_v, [iv])
                        plsc.store_scatter(out8_v, [flat + k], vals)
                    return c1

                lax.fori_loop(0, BSUB // 16, loop, 0)

            pltpu.sync_copy(out8_v, tab_hbm.at[pl.ds((s + j * BSUB) * 8, BSUB * 8)])
        return carry

    lax.fori_loop(0, NCH, chunk, 0)


_build = functools.partial(
    pl.kernel,
    out_type=jax.ShapeDtypeStruct((N * 8,), jnp.float32),
    mesh=plsc.VectorSubcoreMesh(core_axis_name="c", subcore_axis_name="s"),
    scratch_types=[
        pltpu.VMEM((SLAB,), jnp.float32),
        pltpu.VMEM((BSUB * 8,), jnp.float32),
    ],
    compiler_params=pltpu.CompilerParams(needs_layout_passes=False),
)(_build_body)


def _warp_body(ddf0_hbm, ddf1_hbm, ddf2_hbm, tab_hbm, out_hbm,
               dbuf_a, dbuf_b, g_v, wb_v, vals_a, vals_b, out_v,
               gsem, osem, dsem):
    cid = lax.axis_index("c")
    sid = lax.axis_index("s")
    wid = sid * 2 + cid
    lane = lax.broadcasted_iota(jnp.int32, (16,), 0)
    ddf_hbms = (ddf0_hbm, ddf1_hbm, ddf2_hbm)

    def fire_ddf(vb, bufs):
        for c in range(3):
            pltpu.async_copy(ddf_hbms[c].at[pl.ds(vb, C)], bufs[c], dsem)

    def drain_ddf(bufs):
        for c in range(3):
            pltpu.make_async_copy(ddf_hbms[c].at[pl.ds(0, C)], bufs[c], dsem).wait()

    def chunk(ci, dcur, dnxt):
        vb = wid * PER_W + ci * C

        drain_ddf(dcur)
        vb_nxt = jnp.minimum(vb + C, N - C)
        fire_ddf(vb_nxt, dnxt)
        d0_v, d1_v, d2_v = dcur

        def pass1(i, carry1):
            off = i * 16
            v = vb + off + lane
            b = lax.shift_right_logical(v, 21)
            n = jnp.bitwise_and(v, (1 << 21) - 1)
            x = lax.shift_right_logical(n, 14)
            y = jnp.bitwise_and(lax.shift_right_logical(n, 7), 127)
            z = jnp.bitwise_and(n, 127)
            cx = x.astype(jnp.float32) + d0_v[pl.ds(off, 16)]
            cy = y.astype(jnp.float32) + d1_v[pl.ds(off, 16)]
            cz = z.astype(jnp.float32) + d2_v[pl.ds(off, 16)]
            tx = cx.astype(jnp.int32)
            ty = cy.astype(jnp.int32)
            tz = cz.astype(jnp.int32)
            fx = jnp.where(tx.astype(jnp.float32) > cx, tx - 1, tx)
            fy = jnp.where(ty.astype(jnp.float32) > cy, ty - 1, ty)
            fz = jnp.where(tz.astype(jnp.float32) > cz, tz - 1, tz)
            bx = jnp.logical_or(cx >= jnp.float32(D - 1), cx < jnp.float32(0))
            by = jnp.logical_or(cy >= jnp.float32(D - 1), cy < jnp.float32(0))
            bz = jnp.logical_or(cz >= jnp.float32(D - 1), cz < jnp.float32(0))
            zero = jnp.zeros((16,), jnp.float32)
            wb_v[0, pl.ds(off, 16)] = jnp.where(bx, zero, cx - fx.astype(jnp.float32))
            wb_v[1, pl.ds(off, 16)] = jnp.where(by, zero, cy - fy.astype(jnp.float32))
            wb_v[2, pl.ds(off, 16)] = jnp.where(bz, zero, cz - fz.astype(jnp.float32))
            x0 = jnp.clip(fx, 0, D - 1)
            y0 = jnp.clip(fy, 0, D - 1)
            z0 = jnp.clip(fz, 0, D - 1)
            g_v[pl.ds(off, 16)] = (
                lax.shift_left(b, 21)
                + lax.shift_left(x0, 14)
                + lax.shift_left(y0, 7)
                + z0
            )
            return carry1

        lax.fori_loop(0, VPC, pass1, 0)

        @pl.when(ci > 0)
        def _drain_out():
            pltpu.make_async_copy(out_v, out_hbm.at[pl.ds(vb, C)], osem).wait()

        cols = [jnp.full((16,), k, jnp.int32) for k in range(8)]

        def pass2_sub(j, buf):
            def pass2(i, carry2):
                off = j * GSUB + i * 16
                vid = i * 16 + lane
                wxb = wb_v[0, pl.ds(off, 16)]
                wyb = wb_v[1, pl.ds(off, 16)]
                wzb = wb_v[2, pl.ds(off, 16)]
                wxa = 1.0 - wxb
                wya = 1.0 - wyb
                wza = 1.0 - wzb
                w00 = wxa * wya
                w01 = wxa * wyb
                w10 = wxb * wya
                w11 = wxb * wyb
                acc = plsc.load_gather(buf, [vid, cols[0]]) * (w00 * wza)
                acc = acc + plsc.load_gather(buf, [vid, cols[1]]) * (w00 * wzb)
                acc = acc + plsc.load_gather(buf, [vid, cols[2]]) * (w01 * wza)
                acc = acc + plsc.load_gather(buf, [vid, cols[3]]) * (w01 * wzb)
                acc = acc + plsc.load_gather(buf, [vid, cols[4]]) * (w10 * wza)
                acc = acc + plsc.load_gather(buf, [vid, cols[5]]) * (w10 * wzb)
                acc = acc + plsc.load_gather(buf, [vid, cols[6]]) * (w11 * wza)
                acc = acc + plsc.load_gather(buf, [vid, cols[7]]) * (w11 * wzb)
                out_v[pl.ds(off, 16)] = acc
                return carry2

            lax.fori_loop(0, GSUB // 16, pass2, 0)

        def fire(j, buf):
            return pltpu.async_copy(
                tab_hbm.at[g_v.at[pl.ds(j * GSUB, GSUB)]], buf, gsem
            )

        bufs = (vals_a, vals_b)
        subs = C // GSUB
        cp_prev = fire(0, bufs[0])
        for j in range(1, subs):
            cp = fire(j, bufs[j % 2])
            cp_prev.wait()
            pass2_sub(j - 1, bufs[(j - 1) % 2])
            cp_prev = cp
        cp_prev.wait()
        pass2_sub(subs - 1, bufs[(subs - 1) % 2])

        pltpu.async_copy(out_v, out_hbm.at[pl.ds(vb, C)], osem)

    fire_ddf(wid * PER_W, (dbuf_a[0], dbuf_a[1], dbuf_a[2]))

    def pair(t, carry):
        da = (dbuf_a[0], dbuf_a[1], dbuf_a[2])
        db = (dbuf_b[0], dbuf_b[1], dbuf_b[2])
        chunk(2 * t, da, db)
        chunk(2 * t + 1, db, da)
        return carry

    lax.fori_loop(0, NCH // 2, pair, 0)

    drain_ddf((dbuf_a[0], dbuf_a[1], dbuf_a[2]))
    last = wid * PER_W + (NCH - 1) * C
    pltpu.make_async_copy(out_v, out_hbm.at[pl.ds(last, C)], osem).wait()


_warp = functools.partial(
    pl.kernel,
    out_type=jax.ShapeDtypeStruct((N,), jnp.float32),
    mesh=plsc.VectorSubcoreMesh(core_axis_name="c", subcore_axis_name="s"),
    scratch_types=[
        [pltpu.VMEM((C,), jnp.float32) for _ in range(3)],
        [pltpu.VMEM((C,), jnp.float32) for _ in range(3)],
        pltpu.VMEM((C,), jnp.int32),
        pltpu.VMEM((3, C), jnp.float32),
        pltpu.VMEM((GSUB, 8), jnp.float32),
        pltpu.VMEM((GSUB, 8), jnp.float32),
        pltpu.VMEM((C,), jnp.float32),
        pltpu.SemaphoreType.DMA,
        pltpu.SemaphoreType.DMA,
        pltpu.SemaphoreType.DMA,
    ],
    compiler_params=pltpu.CompilerParams(
        needs_layout_passes=False, use_tc_tiling_on_sc=False),
)(_warp_body)


def kernel(ddf, image):
    d0 = ddf[..., 0].reshape(-1)
    d1 = ddf[..., 1].reshape(-1)
    d2 = ddf[..., 2].reshape(-1)
    tab = _build(image.reshape(-1))
    out_flat = _warp(d0, d1, d2, tab.reshape(N, 8))
    return out_flat.reshape(BATCH, D, D, D)


# speedup vs baseline: 4.4879x; 1.1219x over previous
import functools

import jax
import jax.numpy as jnp
from jax import lax
from jax.experimental import pallas as pl
from jax.experimental.pallas import tpu as pltpu
from jax.experimental.pallas import tpu_sc as plsc

D = 128
BATCH = 2
N = BATCH * D * D * D
NW = 32
PER_W = N // NW
C = 4096
NCH = PER_W // C
VPC = C // 16

OFFS = (0, 1, 128, 129, 16384, 16385, 16512, 16513)
SLAB = C + OFFS[-1] + 7
BSUB = 512
GSUB = 256


def _build_body(img_hbm, tab_hbm, slab_a, slab_b, o8a, o8b, ssem, obsem):
    cid = lax.axis_index("c")
    sid = lax.axis_index("s")
    wid = sid * 2 + cid
    lane = lax.broadcasted_iota(jnp.int32, (16,), 0)
    obufs = (o8a, o8b)

    def fire_slab(ci, buf):
        s_eff = jnp.minimum(wid * PER_W + ci * C, N - SLAB)
        pltpu.async_copy(img_hbm.at[pl.ds(s_eff, SLAB)], buf, ssem)

    def drain_slab(buf):
        pltpu.make_async_copy(img_hbm.at[pl.ds(0, SLAB)], buf, ssem).wait()

    def drain_out(buf):
        pltpu.make_async_copy(buf, tab_hbm.at[pl.ds(0, BSUB * 8)], obsem).wait()

    def chunk(ci, slab_v, slab_nxt):
        s = wid * PER_W + ci * C
        delta = s - jnp.minimum(s, N - SLAB)
        drain_slab(slab_v)
        fire_slab(ci + 1, slab_nxt)

        for j in range(C // BSUB):
            ob = obufs[j % 2]
            if j >= 2:
                drain_out(ob)
            else:
                @pl.when(ci > 0)
                def _d():
                    drain_out(ob)

            @pl.when(delta == 0)
            def _fast():
                def loop(i, c1):
                    base = j * BSUB + i * 16
                    flat = (i * 16 + lane) * 8
                    for k in range(8):
                        vals = slab_v[pl.ds(base + OFFS[k], 16)]
                        plsc.store_scatter(ob, [flat + k], vals)
                    return c1

                lax.fori_loop(0, BSUB // 16, loop, 0)

            @pl.when(delta != 0)
            def _clamped():
                def loop(i, c1):
                    base = delta + j * BSUB + i * 16
                    flat = (i * 16 + lane) * 8
                    for k in range(8):
                        iv = jnp.minimum(base + OFFS[k] + lane, SLAB - 1)
                        vals = plsc.load_gather(slab_v, [iv])
                        plsc.store_scatter(ob, [flat + k], vals)
                    return c1

                lax.fori_loop(0, BSUB // 16, loop, 0)

            pltpu.async_copy(
                ob, tab_hbm.at[pl.ds((s + j * BSUB) * 8, BSUB * 8)], obsem)

    fire_slab(0, slab_a)

    def pair(t, carry):
        chunk(2 * t, slab_a, slab_b)
        chunk(2 * t + 1, slab_b, slab_a)
        return carry

    lax.fori_loop(0, NCH // 2, pair, 0)

    drain_slab(slab_a)
    drain_out(o8a)
    drain_out(o8b)


_build = functools.partial(
    pl.kernel,
    out_type=jax.ShapeDtypeStruct((N * 8,), jnp.float32),
    mesh=plsc.VectorSubcoreMesh(core_axis_name="c", subcore_axis_name="s"),
    scratch_types=[
        pltpu.VMEM((SLAB,), jnp.float32),
        pltpu.VMEM((SLAB,), jnp.float32),
        pltpu.VMEM((BSUB * 8,), jnp.float32),
        pltpu.VMEM((BSUB * 8,), jnp.float32),
        pltpu.SemaphoreType.DMA,
        pltpu.SemaphoreType.DMA,
    ],
    compiler_params=pltpu.CompilerParams(needs_layout_passes=False),
)(_build_body)


def _warp_body(ddf0_hbm, ddf1_hbm, ddf2_hbm, tab_hbm, out_hbm,
               dbuf_a, dbuf_b, g_v, wb_v, vals_a, vals_b, out_v,
               gsem, osem, dsem):
    cid = lax.axis_index("c")
    sid = lax.axis_index("s")
    wid = sid * 2 + cid
    lane = lax.broadcasted_iota(jnp.int32, (16,), 0)
    ddf_hbms = (ddf0_hbm, ddf1_hbm, ddf2_hbm)

    def fire_ddf(vb, bufs):
        for c in range(3):
            pltpu.async_copy(ddf_hbms[c].at[pl.ds(vb, C)], bufs[c], dsem)

    def drain_ddf(bufs):
        for c in range(3):
            pltpu.make_async_copy(ddf_hbms[c].at[pl.ds(0, C)], bufs[c], dsem).wait()

    def chunk(ci, dcur, dnxt):
        vb = wid * PER_W + ci * C

        drain_ddf(dcur)
        vb_nxt = jnp.minimum(vb + C, N - C)
        fire_ddf(vb_nxt, dnxt)
        d0_v, d1_v, d2_v = dcur

        def pass1(i, carry1):
            off = i * 16
            v = vb + off + lane
            b = lax.shift_right_logical(v, 21)
            n = jnp.bitwise_and(v, (1 << 21) - 1)
            x = lax.shift_right_logical(n, 14)
            y = jnp.bitwise_and(lax.shift_right_logical(n, 7), 127)
            z = jnp.bitwise_and(n, 127)
            cx = x.astype(jnp.float32) + d0_v[pl.ds(off, 16)]
            cy = y.astype(jnp.float32) + d1_v[pl.ds(off, 16)]
            cz = z.astype(jnp.float32) + d2_v[pl.ds(off, 16)]
            tx = cx.astype(jnp.int32)
            ty = cy.astype(jnp.int32)
            tz = cz.astype(jnp.int32)
            fx = jnp.where(tx.astype(jnp.float32) > cx, tx - 1, tx)
            fy = jnp.where(ty.astype(jnp.float32) > cy, ty - 1, ty)
            fz = jnp.where(tz.astype(jnp.float32) > cz, tz - 1, tz)
            bx = jnp.logical_or(cx >= jnp.float32(D - 1), cx < jnp.float32(0))
            by = jnp.logical_or(cy >= jnp.float32(D - 1), cy < jnp.float32(0))
            bz = jnp.logical_or(cz >= jnp.float32(D - 1), cz < jnp.float32(0))
            zero = jnp.zeros((16,), jnp.float32)
            wb_v[0, pl.ds(off, 16)] = jnp.where(bx, zero, cx - fx.astype(jnp.float32))
            wb_v[1, pl.ds(off, 16)] = jnp.where(by, zero, cy - fy.astype(jnp.float32))
            wb_v[2, pl.ds(off, 16)] = jnp.where(bz, zero, cz - fz.astype(jnp.float32))
            x0 = jnp.clip(fx, 0, D - 1)
            y0 = jnp.clip(fy, 0, D - 1)
            z0 = jnp.clip(fz, 0, D - 1)
            g_v[pl.ds(off, 16)] = (
                lax.shift_left(b, 21)
                + lax.shift_left(x0, 14)
                + lax.shift_left(y0, 7)
                + z0
            )
            return carry1

        lax.fori_loop(0, VPC, pass1, 0)

        @pl.when(ci > 0)
        def _drain_out():
            pltpu.make_async_copy(out_v, out_hbm.at[pl.ds(vb, C)], osem).wait()

        cols = [jnp.full((16,), k, jnp.int32) for k in range(8)]

        def pass2_sub(j, buf):
            def pass2(i, carry2):
                off = j * GSUB + i * 16
                vid = i * 16 + lane
                wxb = wb_v[0, pl.ds(off, 16)]
                wyb = wb_v[1, pl.ds(off, 16)]
                wzb = wb_v[2, pl.ds(off, 16)]
                wxa = 1.0 - wxb
                wya = 1.0 - wyb
                wza = 1.0 - wzb
                w00 = wxa * wya
                w01 = wxa * wyb
                w10 = wxb * wya
                w11 = wxb * wyb
                acc = plsc.load_gather(buf, [vid, cols[0]]) * (w00 * wza)
                acc = acc + plsc.load_gather(buf, [vid, cols[1]]) * (w00 * wzb)
                acc = acc + plsc.load_gather(buf, [vid, cols[2]]) * (w01 * wza)
                acc = acc + plsc.load_gather(buf, [vid, cols[3]]) * (w01 * wzb)
                acc = acc + plsc.load_gather(buf, [vid, cols[4]]) * (w10 * wza)
                acc = acc + plsc.load_gather(buf, [vid, cols[5]]) * (w10 * wzb)
                acc = acc + plsc.load_gather(buf, [vid, cols[6]]) * (w11 * wza)
                acc = acc + plsc.load_gather(buf, [vid, cols[7]]) * (w11 * wzb)
                out_v[pl.ds(off, 16)] = acc
                return carry2

            lax.fori_loop(0, GSUB // 16, pass2, 0)

        def fire(j, buf):
            return pltpu.async_copy(
                tab_hbm.at[g_v.at[pl.ds(j * GSUB, GSUB)]], buf, gsem
            )

        bufs = (vals_a, vals_b)
        subs = C // GSUB
        cp_prev = fire(0, bufs[0])
        for j in range(1, subs):
            cp = fire(j, bufs[j % 2])
            cp_prev.wait()
            pass2_sub(j - 1, bufs[(j - 1) % 2])
            cp_prev = cp
        cp_prev.wait()
        pass2_sub(subs - 1, bufs[(subs - 1) % 2])

        pltpu.async_copy(out_v, out_hbm.at[pl.ds(vb, C)], osem)

    fire_ddf(wid * PER_W, (dbuf_a[0], dbuf_a[1], dbuf_a[2]))

    def pair(t, carry):
        da = (dbuf_a[0], dbuf_a[1], dbuf_a[2])
        db = (dbuf_b[0], dbuf_b[1], dbuf_b[2])
        chunk(2 * t, da, db)
        chunk(2 * t + 1, db, da)
        return carry

    lax.fori_loop(0, NCH // 2, pair, 0)

    drain_ddf((dbuf_a[0], dbuf_a[1], dbuf_a[2]))
    last = wid * PER_W + (NCH - 1) * C
    pltpu.make_async_copy(out_v, out_hbm.at[pl.ds(last, C)], osem).wait()


_warp = functools.partial(
    pl.kernel,
    out_type=jax.ShapeDtypeStruct((N,), jnp.float32),
    mesh=plsc.VectorSubcoreMesh(core_axis_name="c", subcore_axis_name="s"),
    scratch_types=[
        [pltpu.VMEM((C,), jnp.float32) for _ in range(3)],
        [pltpu.VMEM((C,), jnp.float32) for _ in range(3)],
        pltpu.VMEM((C,), jnp.int32),
        pltpu.VMEM((3, C), jnp.float32),
        pltpu.VMEM((GSUB, 8), jnp.float32),
        pltpu.VMEM((GSUB, 8), jnp.float32),
        pltpu.VMEM((C,), jnp.float32),
        pltpu.SemaphoreType.DMA,
        pltpu.SemaphoreType.DMA,
        pltpu.SemaphoreType.DMA,
    ],
    compiler_params=pltpu.CompilerParams(
        needs_layout_passes=False, use_tc_tiling_on_sc=False),
)(_warp_body)


def kernel(ddf, image):
    d0 = ddf[..., 0].reshape(-1)
    d1 = ddf[..., 1].reshape(-1)
    d2 = ddf[..., 2].reshape(-1)
    tab = _build(image.reshape(-1))
    out_flat = _warp(d0, d1, d2, tab.reshape(N, 8))
    return out_flat.reshape(BATCH, D, D, D)
